# baseline (speedup 1.0000x reference)
import functools

import jax
import jax.numpy as jnp
from jax import lax
from jax.experimental import pallas as pl
from jax.experimental.pallas import tpu as pltpu
from jax.experimental.pallas import tpu_sc as plsc

F = 26
B = 16384
D = 64
NC = 2
NS = 16
NW = NC * NS
BPW = B // NW
CHUNK = 128
NCH = BPW // CHUNK
LANES = 16
VECS = D // LANES

_mesh = plsc.VectorSubcoreMesh(core_axis_name="c", subcore_axis_name="s")


@functools.partial(
    pl.kernel,
    out_type=jax.ShapeDtypeStruct((B, D), jnp.float32),
    mesh=_mesh,
    scratch_types=[
        pltpu.VMEM((F * NCH, CHUNK), jnp.int32),
        pltpu.VMEM((BPW, D), jnp.float32),
        pltpu.VMEM((BPW, D), jnp.float32),
        pltpu.SemaphoreType.DMA,
    ],
    compiler_params=pltpu.CompilerParams(use_tc_tiling_on_sc=False),
)
def _encode(idx_hbm, *rest):
    tables = rest[:F]
    out_hbm = rest[F]
    idx_v, acc, tmp, sem = rest[F + 1:]

    wid = lax.axis_index("s") * NC + lax.axis_index("c")
    base = wid * BPW

    pltpu.sync_copy(idx_hbm.at[wid], idx_v)

    def gather_feature(f, dst):
        copies = []
        for c in range(NCH):
            copies.append(
                pltpu.async_copy(
                    tables[f].at[idx_v.at[f * NCH + c]],
                    dst.at[pl.ds(c * CHUNK, CHUNK)],
                    sem,
                )
            )
        for cp in copies:
            cp.wait()

    gather_feature(0, acc)

    def accumulate(_tmp, _acc):
        def body(i, carry):
            for v in range(VECS):
                plsc.addupdate(
                    _acc.at[i, pl.ds(v * LANES, LANES)],
                    _tmp[i, pl.ds(v * LANES, LANES)],
                )
            return carry
        lax.fori_loop(0, BPW, body, 0)

    for f in range(1, F):
        gather_feature(f, tmp)
        accumulate(tmp, acc)

    pltpu.sync_copy(acc, out_hbm.at[pl.ds(base, BPW)])


def kernel(
    cat_0, cat_1, cat_2, cat_3, cat_4, cat_5, cat_6, cat_7, cat_8, cat_9,
    cat_10, cat_11, cat_12, cat_13, cat_14, cat_15, cat_16, cat_17, cat_18,
    cat_19, cat_20, cat_21, cat_22, cat_23, cat_24, cat_25,
    W_cat_0, W_cat_1, W_cat_2, W_cat_3, W_cat_4, W_cat_5, W_cat_6, W_cat_7,
    W_cat_8, W_cat_9, W_cat_10, W_cat_11, W_cat_12, W_cat_13, W_cat_14,
    W_cat_15, W_cat_16, W_cat_17, W_cat_18, W_cat_19, W_cat_20, W_cat_21,
    W_cat_22, W_cat_23, W_cat_24, W_cat_25,
):
    cats = [
        cat_0, cat_1, cat_2, cat_3, cat_4, cat_5, cat_6, cat_7, cat_8,
        cat_9, cat_10, cat_11, cat_12, cat_13, cat_14, cat_15, cat_16,
        cat_17, cat_18, cat_19, cat_20, cat_21, cat_22, cat_23, cat_24,
        cat_25,
    ]
    tables = [
        W_cat_0, W_cat_1, W_cat_2, W_cat_3, W_cat_4, W_cat_5, W_cat_6,
        W_cat_7, W_cat_8, W_cat_9, W_cat_10, W_cat_11, W_cat_12, W_cat_13,
        W_cat_14, W_cat_15, W_cat_16, W_cat_17, W_cat_18, W_cat_19,
        W_cat_20, W_cat_21, W_cat_22, W_cat_23, W_cat_24, W_cat_25,
    ]
    idx = jnp.stack(cats)
    idx = idx.reshape(F, NW, BPW).transpose(1, 0, 2)
    idx = idx.reshape(NW, F * NCH, CHUNK)
    return _encode(idx, *tables)
